# baseline (speedup 1.0000x reference)
import functools

import jax
import jax.numpy as jnp
from jax import lax
from jax.experimental import pallas as pl
from jax.experimental.pallas import tpu as pltpu
from jax.experimental.pallas import tpu_sc as plsc

NBINS = 256
B, C, H, W, D = 4, 32, 64, 64, 64
ROWS = B * C
N = H * W * D
CENTER_H = (N // 2) // (W * D)
NORM = 65 * 65 * 65
LOG2E = 1.4426950408889634


NSTREAM = 4
TC_ROWS = 96
SHARE = TC_ROWS // NSTREAM


def _tc_body(*refs):
    in_refs = refs[:NSTREAM]
    acc_ref, cen_ref = refs[NSTREAM], refs[NSTREAM + 1]
    i = pl.program_id(0)

    for k, ref in enumerate(in_refs):
        row = i + k * SHARE
        blk = ref[0, 0]
        s = blk.sum(axis=0).sum(axis=0, keepdims=True)
        acc_ref[pl.ds(row, 1), :] = s
        cen_ref[pl.ds(row, 1), :] = blk[CENTER_H, 0:1, 0:1]


def _tc_partials(F):
    return pl.pallas_call(
        _tc_body,
        grid=(SHARE,),
        in_specs=[
            pl.BlockSpec(
                (1, 1, H, W, D),
                (lambda i, _k=k: ((i + _k * SHARE) // C, (i + _k * SHARE) % C,
                                  0, 0, 0)),
            )
            for k in range(NSTREAM)
        ],
        out_specs=[
            pl.BlockSpec((TC_ROWS, D), lambda i: (0, 0)),
            pl.BlockSpec((TC_ROWS, 1), lambda i: (0, 0)),
        ],
        out_shape=[
            jax.ShapeDtypeStruct((TC_ROWS, D), jnp.float32),
            jax.ShapeDtypeStruct((TC_ROWS, 1), jnp.float32),
        ],
    )(*([F] * NSTREAM))


NC, NS = 2, 16
NW = NC * NS
SC_ROWS = ROWS - TC_ROWS
CH_H = 2
CPR = H // CH_H
NBUF = 4
OUTW = 32

_mesh = plsc.VectorSubcoreMesh(core_axis_name="c", subcore_axis_name="s")


@functools.partial(
    pl.kernel,
    mesh=_mesh,
    out_type=jax.ShapeDtypeStruct((SC_ROWS * OUTW,), jnp.float32),
    scratch_types=[
        pltpu.VMEM((CH_H, W, D), jnp.float32),
        pltpu.VMEM((CH_H, W, D), jnp.float32),
        pltpu.VMEM((CH_H, W, D), jnp.float32),
        pltpu.VMEM((CH_H, W, D), jnp.float32),
        pltpu.VMEM((16,), jnp.float32),
        pltpu.VMEM((16,), jnp.float32),
        pltpu.SemaphoreType.DMA,
        pltpu.SemaphoreType.DMA,
        pltpu.SemaphoreType.DMA,
        pltpu.SemaphoreType.DMA,
    ],
)
def _sc_rowsum(x_hbm, out_hbm, b0, b1, b2, b3, accv, cenb, s0, s1, s2, s3):
    w = lax.axis_index("c") * NS + lax.axis_index("s")
    row = TC_ROWS + w
    bi = row // C
    ci = row % C
    bufs = (b0, b1, b2, b3)
    sems = (s0, s1, s2, s3)

    accv[...] = jnp.zeros((16,), jnp.float32)

    for t in range(NBUF - 1):
        pltpu.async_copy(
            x_hbm.at[bi, ci, pl.ds(t * CH_H, CH_H)], bufs[t], sems[t]
        )

    for t in range(CPR):
        nxt = t + NBUF - 1
        if nxt < CPR:
            pltpu.async_copy(
                x_hbm.at[bi, ci, pl.ds(nxt * CH_H, CH_H)],
                bufs[nxt % NBUF],
                sems[nxt % NBUF],
            )
        pltpu.make_async_copy(
            x_hbm.at[bi, ci, pl.ds(t * CH_H, CH_H)],
            bufs[t % NBUF],
            sems[t % NBUF],
        ).wait()
        buf = bufs[t % NBUF]

        for h in range(CH_H):

            @pl.loop(0, W)
            def _(wi, _buf=buf, _h=h):
                v = _buf[_h, wi, pl.ds(0, 16)]
                for j in range(1, D // 16):
                    v = v + _buf[_h, wi, pl.ds(16 * j, 16)]
                accv[...] += v

    off = w * OUTW
    pltpu.sync_copy(accv, out_hbm.at[pl.ds(off, 16)])
    pltpu.sync_copy(x_hbm.at[bi, ci, CENTER_H, 0, pl.ds(0, 16)], cenb)
    pltpu.sync_copy(cenb, out_hbm.at[pl.ds(off + 16, 16)])


def _finish_body(acc_ref, cen_ref, sc_ref, out_ref):
    s_tc = acc_ref[...].sum(axis=1, keepdims=True)
    cen_tc = cen_ref[...]
    scp = sc_ref[...]
    s_sc = scp[:, 0:16].sum(axis=1, keepdims=True)
    cen_sc = scp[:, 16:17]
    sums = jnp.concatenate([s_tc, s_sc], axis=0)
    cen = jnp.concatenate([cen_tc, cen_sc], axis=0)
    nb = (sums - cen) * (1.0 / (N - 1))
    vals = jnp.concatenate([cen, nb], axis=0)
    idx = jnp.floor(vals * NBINS).astype(jnp.int32)
    valid = (vals >= 0.0) & (vals <= 1.0)
    idx = jnp.minimum(idx, NBINS - 1)
    bins = lax.broadcasted_iota(jnp.int32, (2 * ROWS, NBINS), 1)
    match = (idx == bins) & valid
    counts = jnp.sum(match.astype(jnp.float32), axis=0, keepdims=True)
    p = counts * (1.0 / NORM)
    e = -jnp.sum(p * (jnp.log(p + 1e-10) * LOG2E), axis=1, keepdims=True)
    out_ref[...] = e


def kernel(F):
    acc, cen = _tc_partials(F)
    scpart = _sc_rowsum(F).reshape(SC_ROWS, OUTW)
    out = pl.pallas_call(
        _finish_body,
        grid=(1,),
        in_specs=[
            pl.BlockSpec((TC_ROWS, D), lambda i: (0, 0)),
            pl.BlockSpec((TC_ROWS, 1), lambda i: (0, 0)),
            pl.BlockSpec((SC_ROWS, OUTW), lambda i: (0, 0)),
        ],
        out_specs=pl.BlockSpec((1, 1), lambda i: (0, 0)),
        out_shape=jax.ShapeDtypeStruct((1, 1), jnp.float32),
    )(acc, cen, scpart)
    return out.reshape(())


# speedup vs baseline: 1.2354x; 1.2354x over previous
import jax
import jax.numpy as jnp
from jax import lax
from jax.experimental import pallas as pl
from jax.experimental.pallas import tpu as pltpu

NBINS = 256
B, C, H, W, D = 4, 32, 64, 64, 64
ROWS = B * C
N = H * W * D
CENTER_H = (N // 2) // (W * D)
NORM = 65 * 65 * 65
LOG2E = 1.4426950408889634

NSTREAM = 4
RPB = 2
SHARE = ROWS // NSTREAM
STEPS = SHARE // RPB


def _entropy_body(*refs):
    in_refs = refs[:NSTREAM]
    out_ref, acc_ref, cen_ref = refs[NSTREAM], refs[NSTREAM + 1], refs[NSTREAM + 2]
    i = pl.program_id(0)

    for k, ref in enumerate(in_refs):
        for r in range(RPB):
            row = i * RPB + r + k * SHARE
            blk = ref[0, r]
            s = blk.sum(axis=0).sum(axis=0, keepdims=True)
            acc_ref[pl.ds(row, 1), :] = s
            cen_ref[pl.ds(row, 1), :] = blk[CENTER_H, 0:1, 0:1]

    @pl.when(i == STEPS - 1)
    def _():
        sums = acc_ref[...].sum(axis=1, keepdims=True)
        cen = cen_ref[...]
        nb = (sums - cen) * (1.0 / (N - 1))
        vals = jnp.concatenate([cen, nb], axis=0)
        idx = jnp.floor(vals * NBINS).astype(jnp.int32)
        valid = (vals >= 0.0) & (vals <= 1.0)
        idx = jnp.minimum(idx, NBINS - 1)
        bins = lax.broadcasted_iota(jnp.int32, (2 * ROWS, NBINS), 1)
        match = (idx == bins) & valid
        counts = jnp.sum(match.astype(jnp.float32), axis=0, keepdims=True)
        p = counts * (1.0 / NORM)
        e = -jnp.sum(p * (jnp.log(p + 1e-10) * LOG2E), axis=1, keepdims=True)
        out_ref[...] = e


def kernel(F):
    out = pl.pallas_call(
        _entropy_body,
        grid=(STEPS,),
        in_specs=[
            pl.BlockSpec(
                (1, RPB, H, W, D),
                (lambda i, _k=k: ((i * RPB + _k * SHARE) // C,
                                  (i * RPB + _k * SHARE) % C // RPB,
                                  0, 0, 0)),
            )
            for k in range(NSTREAM)
        ],
        out_specs=pl.BlockSpec((1, 1), lambda i: (0, 0)),
        out_shape=jax.ShapeDtypeStruct((1, 1), jnp.float32),
        scratch_shapes=[
            pltpu.VMEM((ROWS, D), jnp.float32),
            pltpu.VMEM((ROWS, 1), jnp.float32),
        ],
    )(*([F] * NSTREAM))
    return out.reshape(())


# speedup vs baseline: 1.2551x; 1.0160x over previous
import jax
import jax.numpy as jnp
from jax import lax
from jax.experimental import pallas as pl
from jax.experimental.pallas import tpu as pltpu

NBINS = 256
B, C, H, W, D = 4, 32, 64, 64, 64
ROWS = B * C
N = H * W * D
CENTER_H = (N // 2) // (W * D)
NORM = 65 * 65 * 65
LOG2E = 1.4426950408889634

NSTREAM = 4
RPB = 1
SHARE = ROWS // NSTREAM
STEPS = SHARE // RPB


def _entropy_body(*refs):
    in_refs = refs[:NSTREAM]
    out_ref, acc_ref, cen_ref = refs[NSTREAM], refs[NSTREAM + 1], refs[NSTREAM + 2]
    i = pl.program_id(0)

    for k, ref in enumerate(in_refs):
        for r in range(RPB):
            row = i * RPB + r + k * SHARE
            blk = ref[0, r]
            s = blk.sum(axis=0).sum(axis=0, keepdims=True)
            acc_ref[pl.ds(row, 1), :] = s
            cen_ref[pl.ds(row, 1), :] = blk[CENTER_H, 0:1, 0:1]

    @pl.when(i == STEPS - 1)
    def _():
        sums = acc_ref[...].sum(axis=1, keepdims=True)
        cen = cen_ref[...]
        nb = (sums - cen) * (1.0 / (N - 1))
        vals = jnp.concatenate([cen, nb], axis=0)
        idx = jnp.floor(vals * NBINS).astype(jnp.int32)
        valid = (vals >= 0.0) & (vals <= 1.0)
        idx = jnp.minimum(idx, NBINS - 1)
        bins = lax.broadcasted_iota(jnp.int32, (2 * ROWS, NBINS), 1)
        match = (idx == bins) & valid
        counts = jnp.sum(match.astype(jnp.float32), axis=0, keepdims=True)
        p = counts * (1.0 / NORM)
        e = -jnp.sum(p * (jnp.log(p + 1e-10) * LOG2E), axis=1, keepdims=True)
        out_ref[...] = e


def kernel(F):
    out = pl.pallas_call(
        _entropy_body,
        grid=(STEPS,),
        in_specs=[
            pl.BlockSpec(
                (1, RPB, H, W, D),
                (lambda i, _k=k: ((i * RPB + _k * SHARE) // C,
                                  (i * RPB + _k * SHARE) % C // RPB,
                                  0, 0, 0)),
            )
            for k in range(NSTREAM)
        ],
        out_specs=pl.BlockSpec((1, 1), lambda i: (0, 0)),
        out_shape=jax.ShapeDtypeStruct((1, 1), jnp.float32),
        scratch_shapes=[
            pltpu.VMEM((ROWS, D), jnp.float32),
            pltpu.VMEM((ROWS, 1), jnp.float32),
        ],
    )(*([F] * NSTREAM))
    return out.reshape(())
